# baseline (speedup 1.0000x reference)
import jax
import jax.numpy as jnp
from jax import lax
from jax.experimental import pallas as pl
from jax.experimental.pallas import tpu as pltpu
from jax.experimental.pallas import tpu_sc as plsc

_N, _K, _D = 10000, 32, 128
_NT = 32
_FPT = _D // _NT
_NPAD = 10240
_BN = 512
_NB = _NPAD // _BN
_LANES = 16
_SG = _BN // (2 * _LANES)
_BW = _K * _BN // 2


def _mean_body(tbl, adjb, out, tbl_v, adj_v, out_v, sem_t, sem_a, sem_o):
    cid = lax.axis_index("c")
    sid = lax.axis_index("s")
    wid = sid * 2 + cid

    ct0 = pltpu.async_copy(tbl.at[wid, 0], tbl_v.at[pl.ds(0, _NPAD)], sem_t)
    ct1 = pltpu.async_copy(tbl.at[wid, 1], tbl_v.at[pl.ds(_NPAD, _NPAD)],
                           sem_t)
    pltpu.async_copy(adjb.at[pl.ds(0, _BW)], adj_v.at[pl.ds(0, _BW)], sem_a)
    ct0.wait()
    ct1.wait()
    _mean_core(adjb, out, tbl_v, adj_v, out_v, sem_a, sem_o, wid)


def _mean_body_f32(xt, adjb, out, stg_v, tbl_v, adj_v, out_v,
                   sem_t, sem_a, sem_o):
    cid = lax.axis_index("c")
    sid = lax.axis_index("s")
    wid = sid * 2 + cid

    ct = pltpu.async_copy(xt.at[pl.ds(wid * _FPT, _FPT)], stg_v, sem_t)
    pltpu.async_copy(adjb.at[pl.ds(0, _BW)], adj_v.at[pl.ds(0, _BW)], sem_a)
    ct.wait()

    rnd = jnp.full((_LANES,), 0x8000, jnp.int32)
    msk = jnp.full((_LANES,), -65536, jnp.int32)

    def pack_body(i, _):
        for u in range(16):
            off = (i * 16 + u) * _LANES
            for pr in range(_FPT // 2):
                lo = plsc.bitcast(stg_v[2 * pr, pl.ds(off, _LANES)],
                                  jnp.int32)
                hi = plsc.bitcast(stg_v[2 * pr + 1, pl.ds(off, _LANES)],
                                  jnp.int32)
                w = jnp.bitwise_or(
                    lax.shift_right_logical(lo + rnd, 16),
                    jnp.bitwise_and(hi + rnd, msk))
                tbl_v[pl.ds(pr * _NPAD + off, _LANES)] = w
        return 0

    lax.fori_loop(0, _NPAD // (16 * _LANES), pack_body, 0)
    _mean_core(adjb, out, tbl_v, adj_v, out_v, sem_a, sem_o, wid)


def _mean_core(adjb, out, tbl_v, adj_v, out_v, sem_a, sem_o, wid):
    def block_body(b, _):
        par = jnp.bitwise_and(b, 1)
        pltpu.make_async_copy(
            adjb.at[pl.ds(b * _BW, _BW)],
            adj_v.at[pl.ds(par * _BW, _BW)], sem_a).wait()

        @pl.when(b + 1 < _NB)
        def _():
            pltpu.async_copy(
                adjb.at[pl.ds((b + 1) * _BW, _BW)],
                adj_v.at[pl.ds((1 - par) * _BW, _BW)], sem_a)

        @pl.when(b >= 2)
        def _():
            pltpu.make_async_copy(
                out_v.at[par], out.at[wid, b - 2], sem_o).wait()

        def half_sg(g):
            acc_a = [None] * _FPT
            acc_b = [None] * _FPT
            for k in range(_K):
                w32 = adj_v[pl.ds(par * _BW + k * (_BN // 2) + g * _LANES,
                                  _LANES)]
                ia = jnp.bitwise_and(w32, 0xFFFF)
                ib = lax.shift_right_logical(w32, 16)
                for pr in range(_FPT // 2):
                    wa = plsc.load_gather(tbl_v, [ia + (pr * _NPAD)])
                    wb = plsc.load_gather(tbl_v, [ib + (pr * _NPAD)])
                    vals = (
                        plsc.bitcast(lax.shift_left(wa, 16), jnp.float32),
                        plsc.bitcast(lax.shift_left(wb, 16), jnp.float32),
                        plsc.bitcast(wa, jnp.float32),
                        plsc.bitcast(wb, jnp.float32),
                    )
                    for v, acc, i in (
                            (vals[0], acc_a, 2 * pr),
                            (vals[1], acc_b, 2 * pr),
                            (vals[2], acc_a, 2 * pr + 1),
                            (vals[3], acc_b, 2 * pr + 1)):
                        if k == 0:
                            acc[i] = v
                        else:
                            acc[i] = acc[i] + v
            for f in range(_FPT):
                out_v[par, f, pl.ds(g * 2 * _LANES, _LANES)] = acc_a[f]
                out_v[par, f, pl.ds(g * 2 * _LANES + _LANES, _LANES)] = (
                    acc_b[f])

        def sg_body(g, _):
            half_sg(g)
            return 0

        lax.fori_loop(0, _SG, sg_body, 0)
        pltpu.async_copy(out_v.at[par], out.at[wid, b], sem_o)
        return 0

    lax.fori_loop(0, _NB, block_body, 0)
    pltpu.make_async_copy(out_v.at[0], out.at[wid, _NB - 2], sem_o).wait()
    pltpu.make_async_copy(out_v.at[1], out.at[wid, _NB - 1], sem_o).wait()


def _sc_mean(tbl_slab, adjb):
    mesh = plsc.VectorSubcoreMesh(core_axis_name="c", subcore_axis_name="s")
    return pl.kernel(
        _mean_body,
        out_type=jax.ShapeDtypeStruct((_NT, _NB, _FPT, _BN), jnp.float32),
        mesh=mesh,
        compiler_params=pltpu.CompilerParams(needs_layout_passes=False),
        scratch_types=[
            pltpu.VMEM((_FPT // 2 * _NPAD,), jnp.int32),
            pltpu.VMEM((2 * _BW,), jnp.int32),
            pltpu.VMEM((2, _FPT, _BN), jnp.float32),
            pltpu.SemaphoreType.DMA,
            pltpu.SemaphoreType.DMA,
            pltpu.SemaphoreType.DMA,
        ],
    )(tbl_slab, adjb)


def _sc_mean_f32(xt, adjb):
    mesh = plsc.VectorSubcoreMesh(core_axis_name="c", subcore_axis_name="s")
    return pl.kernel(
        _mean_body_f32,
        out_type=jax.ShapeDtypeStruct((_NT, _NB, _FPT, _BN), jnp.float32),
        mesh=mesh,
        compiler_params=pltpu.CompilerParams(needs_layout_passes=False),
        scratch_types=[
            pltpu.VMEM((_FPT, _NPAD), jnp.float32),
            pltpu.VMEM((_FPT // 2 * _NPAD,), jnp.int32),
            pltpu.VMEM((2 * _BW,), jnp.int32),
            pltpu.VMEM((2, _FPT, _BN), jnp.float32),
            pltpu.SemaphoreType.DMA,
            pltpu.SemaphoreType.DMA,
            pltpu.SemaphoreType.DMA,
        ],
    )(xt, adjb)


def _linear_phases(m_ref, wr_ref):
    m = m_ref[...]
    mt = jnp.concatenate(
        [m[:, j].reshape(_D, _BN) for j in range(4)], axis=1)
    hts = []
    nrm2 = None
    for f in range(_FPT):
        ht = jnp.dot(wr_ref[f], mt, preferred_element_type=jnp.float32)
        ht = jnp.maximum(ht, 0.0)
        hts.append(ht)
        s = jnp.sum(ht * ht, axis=0, keepdims=True)
        nrm2 = s if nrm2 is None else nrm2 + s
    inv = 1.0 / jnp.maximum(jnp.sqrt(nrm2), 1e-12)
    return [ht * inv for ht in hts]


def _linear_packed_body(m_ref, wr_ref, o_ref):
    hts = _linear_phases(m_ref, wr_ref)
    for pr in range(_FPT // 2):
        lo32 = lax.bitcast_convert_type(hts[2 * pr], jnp.uint32)
        hi32 = lax.bitcast_convert_type(hts[2 * pr + 1], jnp.uint32)
        w = jnp.bitwise_or(
            lax.shift_right_logical(lo32 + jnp.uint32(0x8000),
                                    jnp.uint32(16)),
            jnp.bitwise_and(hi32 + jnp.uint32(0x8000),
                            jnp.uint32(0xFFFF0000)))
        o_ref[:, pr, :] = lax.bitcast_convert_type(w, jnp.int32)


def _tc_linear_packed(m, wr):
    return pl.pallas_call(
        _linear_packed_body,
        grid=(_NB // 4,),
        in_specs=[
            pl.BlockSpec((_NT, 4, _FPT, _BN), lambda b: (0, b, 0, 0)),
            pl.BlockSpec((_FPT, _NT, _D), lambda b: (0, 0, 0)),
        ],
        out_specs=pl.BlockSpec((_NT, _FPT // 2, 4 * _BN),
                               lambda b: (0, 0, b)),
        out_shape=jax.ShapeDtypeStruct((_NT, _FPT // 2, _NPAD), jnp.int32),
    )(m, wr)


def _linear_f32_body(m_ref, wr_ref, o_ref):
    hts = _linear_phases(m_ref, wr_ref)
    ht = jnp.stack(hts, axis=1).reshape(_D, 4 * _BN)
    o_ref[...] = ht.T


def _tc_linear_f32(m, wr):
    return pl.pallas_call(
        _linear_f32_body,
        grid=(_NB // 4,),
        in_specs=[
            pl.BlockSpec((_NT, 4, _FPT, _BN), lambda b: (0, b, 0, 0)),
            pl.BlockSpec((_FPT, _NT, _D), lambda b: (0, 0, 0)),
        ],
        out_specs=pl.BlockSpec((4 * _BN, _D), lambda b: (b, 0)),
        out_shape=jax.ShapeDtypeStruct((_NPAD, _D), jnp.float32),
    )(m, wr)


def kernel(x, adj, W0, W1):
    adj_p = jnp.concatenate(
        [adj, jnp.zeros((_NPAD - _N, _K), jnp.int32)], axis=0)
    av = adj_p.astype(jnp.uint32).reshape(_NB, _BN // 32, 2, _LANES, _K)
    aw = jnp.bitwise_or(av[:, :, 0], lax.shift_left(av[:, :, 1],
                                                    jnp.uint32(16)))
    adjb = lax.bitcast_convert_type(
        aw.transpose(0, 3, 1, 2), jnp.int32).reshape(_NB * _BW)
    x_p = jnp.concatenate(
        [x, jnp.zeros((_NPAD - _N, _D), jnp.float32)], axis=0)
    xt = x_p.T

    w0r = W0.reshape(_NT, _FPT, _D).transpose(1, 0, 2) * (1.0 / _K)
    w1r = W1.reshape(_NT, _FPT, _D).transpose(1, 0, 2) * (1.0 / _K)

    m1 = _sc_mean_f32(xt, adjb)
    h1 = _tc_linear_packed(m1, w0r)
    m2 = _sc_mean(h1, adjb)
    h2 = _tc_linear_f32(m2, w1r)
    return h2[:_N]


# speedup vs baseline: 1.0023x; 1.0023x over previous
import jax
import jax.numpy as jnp
from jax import lax
from jax.experimental import pallas as pl
from jax.experimental.pallas import tpu as pltpu
from jax.experimental.pallas import tpu_sc as plsc

_N, _K, _D = 10000, 32, 128
_NT = 32
_FPT = _D // _NT
_NPAD = 10240
_BN = 512
_NB = _NPAD // _BN
_LANES = 16
_SG = _BN // (2 * _LANES)
_BW = _K * _BN // 2


def _mean_body(tbl, adjb, out, tbl_v, adj_v, out_v, sem_t, sem_a, sem_o):
    cid = lax.axis_index("c")
    sid = lax.axis_index("s")
    wid = sid * 2 + cid

    ct0 = pltpu.async_copy(tbl.at[wid, 0], tbl_v.at[pl.ds(0, _NPAD)], sem_t)
    ct1 = pltpu.async_copy(tbl.at[wid, 1], tbl_v.at[pl.ds(_NPAD, _NPAD)],
                           sem_t)
    pltpu.async_copy(adjb.at[pl.ds(0, _BW)], adj_v.at[pl.ds(0, _BW)], sem_a)
    ct0.wait()
    ct1.wait()
    _mean_core(adjb, out, tbl_v, adj_v, out_v, sem_a, sem_o, wid)


def _mean_body_f32(xt, adjb, out, stg_v, tbl_v, adj_v, out_v,
                   sem_t, sem_a, sem_o):
    cid = lax.axis_index("c")
    sid = lax.axis_index("s")
    wid = sid * 2 + cid

    ct = pltpu.async_copy(xt.at[pl.ds(wid * _FPT, _FPT)], stg_v, sem_t)
    pltpu.async_copy(adjb.at[pl.ds(0, _BW)], adj_v.at[pl.ds(0, _BW)], sem_a)
    ct.wait()

    rnd = jnp.full((_LANES,), 0x8000, jnp.int32)
    msk = jnp.full((_LANES,), -65536, jnp.int32)

    def pack_body(i, _):
        for u in range(4):
            off = (i * 4 + u) * _LANES
            for pr in range(_FPT // 2):
                lo = plsc.bitcast(stg_v[2 * pr, pl.ds(off, _LANES)],
                                  jnp.int32)
                hi = plsc.bitcast(stg_v[2 * pr + 1, pl.ds(off, _LANES)],
                                  jnp.int32)
                w = jnp.bitwise_or(
                    lax.shift_right_logical(lo + rnd, 16),
                    jnp.bitwise_and(hi + rnd, msk))
                tbl_v[pl.ds(pr * _NPAD + off, _LANES)] = w
        return 0

    lax.fori_loop(0, _NPAD // (4 * _LANES), pack_body, 0)
    _mean_core(adjb, out, tbl_v, adj_v, out_v, sem_a, sem_o, wid)


def _mean_core(adjb, out, tbl_v, adj_v, out_v, sem_a, sem_o, wid):
    def block_body(b, _):
        par = jnp.bitwise_and(b, 1)
        pltpu.make_async_copy(
            adjb.at[pl.ds(b * _BW, _BW)],
            adj_v.at[pl.ds(par * _BW, _BW)], sem_a).wait()

        @pl.when(b + 1 < _NB)
        def _():
            pltpu.async_copy(
                adjb.at[pl.ds((b + 1) * _BW, _BW)],
                adj_v.at[pl.ds((1 - par) * _BW, _BW)], sem_a)

        @pl.when(b >= 2)
        def _():
            pltpu.make_async_copy(
                out_v.at[par], out.at[wid, b - 2], sem_o).wait()

        def half_sg(g):
            acc_a = [None] * _FPT
            acc_b = [None] * _FPT
            for k in range(_K):
                w32 = adj_v[pl.ds(par * _BW + k * (_BN // 2) + g * _LANES,
                                  _LANES)]
                ia = jnp.bitwise_and(w32, 0xFFFF)
                ib = lax.shift_right_logical(w32, 16)
                for pr in range(_FPT // 2):
                    wa = plsc.load_gather(tbl_v, [ia + (pr * _NPAD)])
                    wb = plsc.load_gather(tbl_v, [ib + (pr * _NPAD)])
                    vals = (
                        plsc.bitcast(lax.shift_left(wa, 16), jnp.float32),
                        plsc.bitcast(lax.shift_left(wb, 16), jnp.float32),
                        plsc.bitcast(wa, jnp.float32),
                        plsc.bitcast(wb, jnp.float32),
                    )
                    for v, acc, i in (
                            (vals[0], acc_a, 2 * pr),
                            (vals[1], acc_b, 2 * pr),
                            (vals[2], acc_a, 2 * pr + 1),
                            (vals[3], acc_b, 2 * pr + 1)):
                        if k == 0:
                            acc[i] = v
                        else:
                            acc[i] = acc[i] + v
            for f in range(_FPT):
                out_v[par, f, pl.ds(g * 2 * _LANES, _LANES)] = acc_a[f]
                out_v[par, f, pl.ds(g * 2 * _LANES + _LANES, _LANES)] = (
                    acc_b[f])

        def sg_body(g, _):
            half_sg(g)
            return 0

        lax.fori_loop(0, _SG, sg_body, 0)
        pltpu.async_copy(out_v.at[par], out.at[wid, b], sem_o)
        return 0

    lax.fori_loop(0, _NB, block_body, 0)
    pltpu.make_async_copy(out_v.at[0], out.at[wid, _NB - 2], sem_o).wait()
    pltpu.make_async_copy(out_v.at[1], out.at[wid, _NB - 1], sem_o).wait()


def _sc_mean(tbl_slab, adjb):
    mesh = plsc.VectorSubcoreMesh(core_axis_name="c", subcore_axis_name="s")
    return pl.kernel(
        _mean_body,
        out_type=jax.ShapeDtypeStruct((_NT, _NB, _FPT, _BN), jnp.float32),
        mesh=mesh,
        compiler_params=pltpu.CompilerParams(needs_layout_passes=False),
        scratch_types=[
            pltpu.VMEM((_FPT // 2 * _NPAD,), jnp.int32),
            pltpu.VMEM((2 * _BW,), jnp.int32),
            pltpu.VMEM((2, _FPT, _BN), jnp.float32),
            pltpu.SemaphoreType.DMA,
            pltpu.SemaphoreType.DMA,
            pltpu.SemaphoreType.DMA,
        ],
    )(tbl_slab, adjb)


def _sc_mean_f32(xt, adjb):
    mesh = plsc.VectorSubcoreMesh(core_axis_name="c", subcore_axis_name="s")
    return pl.kernel(
        _mean_body_f32,
        out_type=jax.ShapeDtypeStruct((_NT, _NB, _FPT, _BN), jnp.float32),
        mesh=mesh,
        compiler_params=pltpu.CompilerParams(needs_layout_passes=False),
        scratch_types=[
            pltpu.VMEM((_FPT, _NPAD), jnp.float32),
            pltpu.VMEM((_FPT // 2 * _NPAD,), jnp.int32),
            pltpu.VMEM((2 * _BW,), jnp.int32),
            pltpu.VMEM((2, _FPT, _BN), jnp.float32),
            pltpu.SemaphoreType.DMA,
            pltpu.SemaphoreType.DMA,
            pltpu.SemaphoreType.DMA,
        ],
    )(xt, adjb)


def _linear_phases(m_ref, wr_ref):
    m = m_ref[...]
    mt = jnp.concatenate(
        [m[:, j].reshape(_D, _BN) for j in range(4)], axis=1)
    hts = []
    nrm2 = None
    for f in range(_FPT):
        ht = jnp.dot(wr_ref[f], mt, preferred_element_type=jnp.float32)
        ht = jnp.maximum(ht, 0.0)
        hts.append(ht)
        s = jnp.sum(ht * ht, axis=0, keepdims=True)
        nrm2 = s if nrm2 is None else nrm2 + s
    inv = 1.0 / jnp.maximum(jnp.sqrt(nrm2), 1e-12)
    return [ht * inv for ht in hts]


def _linear_packed_body(m_ref, wr_ref, o_ref):
    hts = _linear_phases(m_ref, wr_ref)
    for pr in range(_FPT // 2):
        lo32 = lax.bitcast_convert_type(hts[2 * pr], jnp.uint32)
        hi32 = lax.bitcast_convert_type(hts[2 * pr + 1], jnp.uint32)
        w = jnp.bitwise_or(
            lax.shift_right_logical(lo32 + jnp.uint32(0x8000),
                                    jnp.uint32(16)),
            jnp.bitwise_and(hi32 + jnp.uint32(0x8000),
                            jnp.uint32(0xFFFF0000)))
        o_ref[:, pr, :] = lax.bitcast_convert_type(w, jnp.int32)


def _tc_linear_packed(m, wr):
    return pl.pallas_call(
        _linear_packed_body,
        grid=(_NB // 4,),
        in_specs=[
            pl.BlockSpec((_NT, 4, _FPT, _BN), lambda b: (0, b, 0, 0)),
            pl.BlockSpec((_FPT, _NT, _D), lambda b: (0, 0, 0)),
        ],
        out_specs=pl.BlockSpec((_NT, _FPT // 2, 4 * _BN),
                               lambda b: (0, 0, b)),
        out_shape=jax.ShapeDtypeStruct((_NT, _FPT // 2, _NPAD), jnp.int32),
    )(m, wr)


def _linear_f32_body(m_ref, wr_ref, o_ref):
    hts = _linear_phases(m_ref, wr_ref)
    ht = jnp.stack(hts, axis=1).reshape(_D, 4 * _BN)
    o_ref[...] = ht.T


def _tc_linear_f32(m, wr):
    return pl.pallas_call(
        _linear_f32_body,
        grid=(_NB // 4,),
        in_specs=[
            pl.BlockSpec((_NT, 4, _FPT, _BN), lambda b: (0, b, 0, 0)),
            pl.BlockSpec((_FPT, _NT, _D), lambda b: (0, 0, 0)),
        ],
        out_specs=pl.BlockSpec((4 * _BN, _D), lambda b: (b, 0)),
        out_shape=jax.ShapeDtypeStruct((_NPAD, _D), jnp.float32),
    )(m, wr)


def kernel(x, adj, W0, W1):
    adj_p = jnp.concatenate(
        [adj, jnp.zeros((_NPAD - _N, _K), jnp.int32)], axis=0)
    av = adj_p.astype(jnp.uint32).reshape(_NB, _BN // 32, 2, _LANES, _K)
    aw = jnp.bitwise_or(av[:, :, 0], lax.shift_left(av[:, :, 1],
                                                    jnp.uint32(16)))
    adjb = lax.bitcast_convert_type(
        aw.transpose(0, 3, 1, 2), jnp.int32).reshape(_NB * _BW)
    x_p = jnp.concatenate(
        [x, jnp.zeros((_NPAD - _N, _D), jnp.float32)], axis=0)
    xt = x_p.T

    w0r = W0.reshape(_NT, _FPT, _D).transpose(1, 0, 2) * (1.0 / _K)
    w1r = W1.reshape(_NT, _FPT, _D).transpose(1, 0, 2) * (1.0 / _K)

    m1 = _sc_mean_f32(xt, adjb)
    h1 = _tc_linear_packed(m1, w0r)
    m2 = _sc_mean(h1, adjb)
    h2 = _tc_linear_f32(m2, w1r)
    return h2[:_N]


# speedup vs baseline: 1.0338x; 1.0314x over previous
import jax
import jax.numpy as jnp
from jax import lax
from jax.experimental import pallas as pl
from jax.experimental.pallas import tpu as pltpu
from jax.experimental.pallas import tpu_sc as plsc

_N, _K, _D = 10000, 32, 128
_NT = 32
_FPT = _D // _NT
_NPAD = 10240
_BN = 512
_NB = _NPAD // _BN
_LANES = 16
_SG = _BN // (2 * _LANES)
_BW = _K * _BN // 2


def _mean_body(tbl, adjb, out, tbl_v, adj_v, out_v, sem_t, sem_a, sem_o):
    cid = lax.axis_index("c")
    sid = lax.axis_index("s")
    wid = sid * 2 + cid

    ct0 = pltpu.async_copy(tbl.at[wid, 0], tbl_v.at[pl.ds(0, _NPAD)], sem_t)
    ct1 = pltpu.async_copy(tbl.at[wid, 1], tbl_v.at[pl.ds(_NPAD, _NPAD)],
                           sem_t)
    pltpu.async_copy(adjb.at[pl.ds(0, _BW)], adj_v.at[pl.ds(0, _BW)], sem_a)
    ct0.wait()
    ct1.wait()
    _mean_core(adjb, out, tbl_v, adj_v, out_v, sem_a, sem_o, wid)


def _mean_body_f32(xt, adjb, out, stg_v, tbl_v, adj_v, out_v,
                   sem_t, sem_a, sem_o):
    cid = lax.axis_index("c")
    sid = lax.axis_index("s")
    wid = sid * 2 + cid

    ct = pltpu.async_copy(xt.at[pl.ds(wid * _FPT, _FPT)], stg_v, sem_t)
    pltpu.async_copy(adjb.at[pl.ds(0, _BW)], adj_v.at[pl.ds(0, _BW)], sem_a)
    ct.wait()

    rnd = jnp.full((_LANES,), 0x8000, jnp.int32)
    msk = jnp.full((_LANES,), -65536, jnp.int32)

    @plsc.parallel_loop(0, _NPAD // (4 * _LANES), step=1)
    def pack_body(i):
        for u in range(4):
            off = (i * 4 + u) * _LANES
            for pr in range(_FPT // 2):
                lo = plsc.bitcast(stg_v[2 * pr, pl.ds(off, _LANES)],
                                  jnp.int32)
                hi = plsc.bitcast(stg_v[2 * pr + 1, pl.ds(off, _LANES)],
                                  jnp.int32)
                w = jnp.bitwise_or(
                    lax.shift_right_logical(lo + rnd, 16),
                    jnp.bitwise_and(hi + rnd, msk))
                tbl_v[pl.ds(pr * _NPAD + off, _LANES)] = w
    _mean_core(adjb, out, tbl_v, adj_v, out_v, sem_a, sem_o, wid)


def _mean_core(adjb, out, tbl_v, adj_v, out_v, sem_a, sem_o, wid):
    def block_body(b, _):
        par = jnp.bitwise_and(b, 1)
        pltpu.make_async_copy(
            adjb.at[pl.ds(b * _BW, _BW)],
            adj_v.at[pl.ds(par * _BW, _BW)], sem_a).wait()

        @pl.when(b + 1 < _NB)
        def _():
            pltpu.async_copy(
                adjb.at[pl.ds((b + 1) * _BW, _BW)],
                adj_v.at[pl.ds((1 - par) * _BW, _BW)], sem_a)

        @pl.when(b >= 2)
        def _():
            pltpu.make_async_copy(
                out_v.at[par], out.at[wid, b - 2], sem_o).wait()

        def half_sg(g):
            acc_a = [None] * _FPT
            acc_b = [None] * _FPT
            for k in range(_K):
                w32 = adj_v[pl.ds(par * _BW + k * (_BN // 2) + g * _LANES,
                                  _LANES)]
                ia = jnp.bitwise_and(w32, 0xFFFF)
                ib = lax.shift_right_logical(w32, 16)
                for pr in range(_FPT // 2):
                    wa = plsc.load_gather(tbl_v, [ia + (pr * _NPAD)])
                    wb = plsc.load_gather(tbl_v, [ib + (pr * _NPAD)])
                    vals = (
                        plsc.bitcast(lax.shift_left(wa, 16), jnp.float32),
                        plsc.bitcast(lax.shift_left(wb, 16), jnp.float32),
                        plsc.bitcast(wa, jnp.float32),
                        plsc.bitcast(wb, jnp.float32),
                    )
                    for v, acc, i in (
                            (vals[0], acc_a, 2 * pr),
                            (vals[1], acc_b, 2 * pr),
                            (vals[2], acc_a, 2 * pr + 1),
                            (vals[3], acc_b, 2 * pr + 1)):
                        if k == 0:
                            acc[i] = v
                        else:
                            acc[i] = acc[i] + v
            for f in range(_FPT):
                out_v[par, f, pl.ds(g * 2 * _LANES, _LANES)] = acc_a[f]
                out_v[par, f, pl.ds(g * 2 * _LANES + _LANES, _LANES)] = (
                    acc_b[f])

        def sg_body(g, _):
            half_sg(g)
            return 0

        lax.fori_loop(0, _SG, sg_body, 0)
        pltpu.async_copy(out_v.at[par], out.at[wid, b], sem_o)
        return 0

    lax.fori_loop(0, _NB, block_body, 0)
    pltpu.make_async_copy(out_v.at[0], out.at[wid, _NB - 2], sem_o).wait()
    pltpu.make_async_copy(out_v.at[1], out.at[wid, _NB - 1], sem_o).wait()


def _sc_mean(tbl_slab, adjb):
    mesh = plsc.VectorSubcoreMesh(core_axis_name="c", subcore_axis_name="s")
    return pl.kernel(
        _mean_body,
        out_type=jax.ShapeDtypeStruct((_NT, _NB, _FPT, _BN), jnp.float32),
        mesh=mesh,
        compiler_params=pltpu.CompilerParams(needs_layout_passes=False),
        scratch_types=[
            pltpu.VMEM((_FPT // 2 * _NPAD,), jnp.int32),
            pltpu.VMEM((2 * _BW,), jnp.int32),
            pltpu.VMEM((2, _FPT, _BN), jnp.float32),
            pltpu.SemaphoreType.DMA,
            pltpu.SemaphoreType.DMA,
            pltpu.SemaphoreType.DMA,
        ],
    )(tbl_slab, adjb)


def _sc_mean_f32(xt, adjb):
    mesh = plsc.VectorSubcoreMesh(core_axis_name="c", subcore_axis_name="s")
    return pl.kernel(
        _mean_body_f32,
        out_type=jax.ShapeDtypeStruct((_NT, _NB, _FPT, _BN), jnp.float32),
        mesh=mesh,
        compiler_params=pltpu.CompilerParams(needs_layout_passes=False),
        scratch_types=[
            pltpu.VMEM((_FPT, _NPAD), jnp.float32),
            pltpu.VMEM((_FPT // 2 * _NPAD,), jnp.int32),
            pltpu.VMEM((2 * _BW,), jnp.int32),
            pltpu.VMEM((2, _FPT, _BN), jnp.float32),
            pltpu.SemaphoreType.DMA,
            pltpu.SemaphoreType.DMA,
            pltpu.SemaphoreType.DMA,
        ],
    )(xt, adjb)


def _linear_phases(m_ref, wr_ref):
    m = m_ref[...]
    mt = jnp.concatenate(
        [m[:, j].reshape(_D, _BN) for j in range(4)], axis=1)
    hts = []
    nrm2 = None
    for f in range(_FPT):
        ht = jnp.dot(wr_ref[f], mt, preferred_element_type=jnp.float32)
        ht = jnp.maximum(ht, 0.0)
        hts.append(ht)
        s = jnp.sum(ht * ht, axis=0, keepdims=True)
        nrm2 = s if nrm2 is None else nrm2 + s
    inv = 1.0 / jnp.maximum(jnp.sqrt(nrm2), 1e-12)
    return [ht * inv for ht in hts]


def _linear_packed_body(m_ref, wr_ref, o_ref):
    hts = _linear_phases(m_ref, wr_ref)
    for pr in range(_FPT // 2):
        lo32 = lax.bitcast_convert_type(hts[2 * pr], jnp.uint32)
        hi32 = lax.bitcast_convert_type(hts[2 * pr + 1], jnp.uint32)
        w = jnp.bitwise_or(
            lax.shift_right_logical(lo32 + jnp.uint32(0x8000),
                                    jnp.uint32(16)),
            jnp.bitwise_and(hi32 + jnp.uint32(0x8000),
                            jnp.uint32(0xFFFF0000)))
        o_ref[:, pr, :] = lax.bitcast_convert_type(w, jnp.int32)


def _tc_linear_packed(m, wr):
    return pl.pallas_call(
        _linear_packed_body,
        grid=(_NB // 4,),
        in_specs=[
            pl.BlockSpec((_NT, 4, _FPT, _BN), lambda b: (0, b, 0, 0)),
            pl.BlockSpec((_FPT, _NT, _D), lambda b: (0, 0, 0)),
        ],
        out_specs=pl.BlockSpec((_NT, _FPT // 2, 4 * _BN),
                               lambda b: (0, 0, b)),
        out_shape=jax.ShapeDtypeStruct((_NT, _FPT // 2, _NPAD), jnp.int32),
    )(m, wr)


def _linear_f32_body(m_ref, wr_ref, o_ref):
    hts = _linear_phases(m_ref, wr_ref)
    ht = jnp.stack(hts, axis=1).reshape(_D, 4 * _BN)
    o_ref[...] = ht.T


def _tc_linear_f32(m, wr):
    return pl.pallas_call(
        _linear_f32_body,
        grid=(_NB // 4,),
        in_specs=[
            pl.BlockSpec((_NT, 4, _FPT, _BN), lambda b: (0, b, 0, 0)),
            pl.BlockSpec((_FPT, _NT, _D), lambda b: (0, 0, 0)),
        ],
        out_specs=pl.BlockSpec((4 * _BN, _D), lambda b: (b, 0)),
        out_shape=jax.ShapeDtypeStruct((_NPAD, _D), jnp.float32),
    )(m, wr)


def kernel(x, adj, W0, W1):
    adj_p = jnp.concatenate(
        [adj, jnp.zeros((_NPAD - _N, _K), jnp.int32)], axis=0)
    av = adj_p.astype(jnp.uint32).reshape(_NB, _BN // 32, 2, _LANES, _K)
    aw = jnp.bitwise_or(av[:, :, 0], lax.shift_left(av[:, :, 1],
                                                    jnp.uint32(16)))
    adjb = lax.bitcast_convert_type(
        aw.transpose(0, 3, 1, 2), jnp.int32).reshape(_NB * _BW)
    x_p = jnp.concatenate(
        [x, jnp.zeros((_NPAD - _N, _D), jnp.float32)], axis=0)
    xt = x_p.T

    w0r = W0.reshape(_NT, _FPT, _D).transpose(1, 0, 2) * (1.0 / _K)
    w1r = W1.reshape(_NT, _FPT, _D).transpose(1, 0, 2) * (1.0 / _K)

    m1 = _sc_mean_f32(xt, adjb)
    h1 = _tc_linear_packed(m1, w0r)
    m2 = _sc_mean(h1, adjb)
    h2 = _tc_linear_f32(m2, w1r)
    return h2[:_N]
